# baseline (speedup 1.0000x reference)
import functools

import jax
import jax.numpy as jnp
from jax import lax
from jax.experimental import pallas as pl
from jax.experimental.pallas import tpu as pltpu
from jax.experimental.pallas import tpu_sc as plsc

NJ = 50
NI = 16384
DIM = 64
NC, NS = 2, 16
NW = NC * NS
LANES = 128
NGROUPS = NJ * (NI // LANES)
GPW = NGROUPS // NW

_mesh = plsc.VectorSubcoreMesh(core_axis_name="c", subcore_axis_name="s")


@functools.partial(
    pl.kernel,
    out_type=jax.ShapeDtypeStruct((NJ, DIM, NI), jnp.float32),
    mesh=_mesh,
    scratch_types=[
        pltpu.VMEM((LANES,), jnp.int32),
        pltpu.VMEM((2, LANES, LANES), jnp.float32),
        pltpu.VMEM((2, LANES), jnp.int32),
        pltpu.VMEM((2, DIM, LANES), jnp.float32),
        pltpu.VMEM((2, LANES), jnp.int32),
        pltpu.SemaphoreType.DMA,
        pltpu.SemaphoreType.DMA,
    ],
    compiler_params=pltpu.CompilerParams(
        use_tc_tiling_on_sc=True, needs_layout_passes=False),
)
def _emb_lookup(ids_hbm, table_hbm, out_hbm, idsv, gbuf, gidx, otile, cbase,
                gsem, osem):
    wid = lax.axis_index("s") * NC + lax.axis_index("c")
    g0 = wid * GPW
    iota = lax.iota(jnp.int32, 16)

    def load_ids_and_fire(item, b):
        j = item // (NI // LANES)
        c = item % (NI // LANES)
        pltpu.sync_copy(ids_hbm.at[j, pl.ds(c * LANES, LANES)], idsv)
        for k in range(8):
            v = idsv[pl.ds(16 * k, 16)]
            gidx.at[b][pl.ds(16 * k, 16)] = lax.shift_right_logical(v, 1)
            cbase.at[b][pl.ds(16 * k, 16)] = lax.shift_left(
                lax.bitwise_and(v, 1), 6)
        pltpu.async_copy(table_hbm.at[gidx.at[b]], gbuf.at[b], gsem)

    def wait_gather(b):
        pltpu.make_async_copy(
            table_hbm.at[pl.ds(0, LANES)], gbuf.at[b], gsem).wait()

    def drain_stores(b):
        for r in range(8):
            pltpu.make_async_copy(
                otile.at[b].at[pl.ds(r * 8, 8)],
                out_hbm.at[0, pl.ds(0, 8), pl.ds(0, LANES)],
                osem).wait()

    def transpose_and_store(item, b, g):
        j = item // (NI // LANES)
        c = item % (NI // LANES)
        G = gbuf.at[b]
        cb = [cbase.at[b][pl.ds(16 * lb, 16)] for lb in range(8)]
        li = [iota + (16 * lb) for lb in range(8)]

        @pl.when(g > 0)
        def _():
            drain_stores(b)
        ot = otile.at[b]

        @plsc.parallel_loop(0, DIM, unroll=8)
        def _d(d):
            for lb in range(8):
                val = plsc.load_gather(G, [li[lb], cb[lb] + d])
                ot[d, pl.ds(16 * lb, 16)] = val

        for r in range(8):
            pltpu.async_copy(
                ot.at[pl.ds(r * 8, 8)],
                out_hbm.at[j, pl.ds(r * 8, 8), pl.ds(c * LANES, LANES)],
                osem)

    load_ids_and_fire(g0, 0)

    @pl.loop(0, GPW, step=2)
    def _g(g):
        item = g0 + g
        wait_gather(0)
        load_ids_and_fire(item + 1, 1)
        transpose_and_store(item, 0, g)
        wait_gather(1)

        @pl.when(g + 2 < GPW)
        def _():
            load_ids_and_fire(item + 2, 0)
        transpose_and_store(item + 1, 1, g)

    drain_stores(0)
    drain_stores(1)


def kernel(token_ids, W):
    ids_t = token_ids.T.astype(jnp.int32)
    table = W.reshape(500000, 128)
    out = _emb_lookup(ids_t, table)
    return out.transpose(2, 0, 1)


# speedup vs baseline: 1.0802x; 1.0802x over previous
import functools

import jax
import jax.numpy as jnp
from jax import lax
from jax.experimental import pallas as pl
from jax.experimental.pallas import tpu as pltpu
from jax.experimental.pallas import tpu_sc as plsc

NJ = 50
NI = 16384
DIM = 64
NC, NS = 2, 16
NW = NC * NS
LANES = 128
NGROUPS = NJ * (NI // LANES)
GPW = NGROUPS // NW

_mesh = plsc.VectorSubcoreMesh(core_axis_name="c", subcore_axis_name="s")


@functools.partial(
    pl.kernel,
    out_type=jax.ShapeDtypeStruct((NJ, DIM, NI), jnp.float32),
    mesh=_mesh,
    scratch_types=[
        pltpu.VMEM((2, LANES), jnp.int32),
        pltpu.VMEM((2, LANES, LANES), jnp.float32),
        pltpu.VMEM((2, LANES), jnp.int32),
        pltpu.VMEM((2, DIM, LANES), jnp.float32),
        pltpu.VMEM((2, LANES), jnp.int32),
        pltpu.SemaphoreType.DMA,
        pltpu.SemaphoreType.DMA,
        pltpu.SemaphoreType.DMA,
    ],
    compiler_params=pltpu.CompilerParams(
        use_tc_tiling_on_sc=True, needs_layout_passes=False),
)
def _emb_lookup(ids_hbm, table_hbm, out_hbm, idsv, gbuf, gidx, otile, cbase,
                gsem, osem, isem):
    wid = lax.axis_index("s") * NC + lax.axis_index("c")
    g0 = wid * GPW
    iota = lax.iota(jnp.int32, 16)

    def fire_ids(item, b):
        j = item // (NI // LANES)
        c = item % (NI // LANES)
        pltpu.async_copy(
            ids_hbm.at[j, pl.ds(c * LANES, LANES)], idsv.at[b], isem)

    def wait_ids(b):
        pltpu.make_async_copy(
            ids_hbm.at[0, pl.ds(0, LANES)], idsv.at[b], isem).wait()

    def fire_gather(b):
        for k in range(8):
            v = idsv.at[b][pl.ds(16 * k, 16)]
            gidx.at[b][pl.ds(16 * k, 16)] = lax.shift_right_logical(v, 1)
            cbase.at[b][pl.ds(16 * k, 16)] = lax.shift_left(
                lax.bitwise_and(v, 1), 6)
        pltpu.async_copy(table_hbm.at[gidx.at[b]], gbuf.at[b], gsem)

    def wait_gather(b):
        pltpu.make_async_copy(
            table_hbm.at[pl.ds(0, LANES)], gbuf.at[b], gsem).wait()

    def drain_stores(b):
        for r in range(8):
            pltpu.make_async_copy(
                otile.at[b].at[pl.ds(r * 8, 8)],
                out_hbm.at[0, pl.ds(0, 8), pl.ds(0, LANES)],
                osem).wait()

    def transpose_and_store(item, b, g):
        j = item // (NI // LANES)
        c = item % (NI // LANES)
        G = gbuf.at[b]
        cb = [cbase.at[b][pl.ds(16 * lb, 16)] for lb in range(8)]
        li = [iota + (16 * lb) for lb in range(8)]

        @pl.when(g > 0)
        def _():
            drain_stores(b)
        ot = otile.at[b]

        @plsc.parallel_loop(0, DIM, unroll=8)
        def _d(d):
            for lb in range(8):
                val = plsc.load_gather(G, [li[lb], cb[lb] + d])
                ot[d, pl.ds(16 * lb, 16)] = val

        for r in range(8):
            pltpu.async_copy(
                ot.at[pl.ds(r * 8, 8)],
                out_hbm.at[j, pl.ds(r * 8, 8), pl.ds(c * LANES, LANES)],
                osem)

    fire_ids(g0, 0)
    wait_ids(0)
    fire_gather(0)
    fire_ids(g0 + 1, 1)

    @pl.loop(0, GPW, step=2)
    def _g(g):
        item = g0 + g
        wait_gather(0)
        wait_ids(1)
        fire_gather(1)

        @pl.when(g + 2 < GPW)
        def _():
            fire_ids(item + 2, 0)
        transpose_and_store(item, 0, g)
        wait_gather(1)

        @pl.when(g + 2 < GPW)
        def _():
            wait_ids(0)
            fire_gather(0)

        @pl.when(g + 3 < GPW)
        def _():
            fire_ids(item + 3, 1)
        transpose_and_store(item + 1, 1, g)

    drain_stores(0)
    drain_stores(1)


def kernel(token_ids, W):
    ids_t = token_ids.T.astype(jnp.int32)
    table = W.reshape(500000, 128)
    out = _emb_lookup(ids_t, table)
    return out.transpose(2, 0, 1)
